# baseline (speedup 1.0000x reference)
import functools

import jax
import jax.numpy as jnp
from jax import lax
from jax.experimental import pallas as pl
from jax.experimental.pallas import tpu as pltpu
from jax.experimental.pallas import tpu_sc as plsc

NC = 2
NS = 16
L = 16


def _sc_aggregate(feat, src, dst, et, num_rels):
    N, F = feat.shape
    E = et.shape[0]
    RPC = num_rels // NC
    CH = E // NS
    SUB = 1024
    NSUB = CH // SUB
    VPS = SUB // L
    BATCH = 64
    NB = CH // BATCH
    NPAIR = NB // 2
    SHIFT_D = 14
    SHIFT_E = 28
    MASK14 = (1 << 14) - 1
    ZSL = -(-(N + 1) // (NS * 8)) * 8
    N_SP = ZSL * NS
    WB = (N // (NS * 8)) * 8
    WB_LAST = N - (NS - 1) * WB
    DUMP = N

    mesh = plsc.VectorSubcoreMesh(core_axis_name="c", subcore_axis_name="s")

    @functools.partial(
        pl.kernel,
        out_type=jax.ShapeDtypeStruct((num_rels * N, F), jnp.float32),
        mesh=mesh,
        compiler_params=pltpu.CompilerParams(use_tc_tiling_on_sc=False),
        scratch_types=[
            pltpu.VMEM((SUB,), jnp.int32),
            pltpu.VMEM((SUB,), jnp.int32),
            pltpu.VMEM((SUB,), jnp.int32),
            pltpu.VMEM((CH + BATCH,), jnp.int32),
            pltpu.VMEM((BATCH,), jnp.int32),
            pltpu.VMEM((BATCH,), jnp.int32),
            pltpu.VMEM((1, BATCH), jnp.int32),
            pltpu.VMEM((1, BATCH), jnp.int32),
            pltpu.VMEM((BATCH, F), jnp.float32),
            pltpu.VMEM((BATCH, F), jnp.float32),
            pltpu.VMEM((32, F), jnp.float32),
            pltpu.VMEM_SHARED((N_SP, F), jnp.float32),
            pltpu.SemaphoreType.DMA,
            pltpu.SemaphoreType.DMA,
        ],
    )
    def sc_kernel(feat_h, src_h, dst_h, et_h, out_h,
                  et_s, src_s, dst_s, plist, gidx0, gidx1, didx0, didx1,
                  fbuf0, fbuf1, zbuf, agg, sem0, sem1):
        c = lax.axis_index("c")
        s = lax.axis_index("s")
        base_e = s * CH
        rel0 = c * RPC

        zv = jnp.zeros((L,), jnp.float32)
        dummyv = jnp.full((L,), DUMP << SHIFT_D, jnp.int32)
        spl = F // L

        def zb_body(k, _):
            zbuf[k // spl, pl.ds((k % spl) * L, L)] = zv
            return 0
        lax.fori_loop(0, 32 * spl, zb_body, 0)
        for q in range(BATCH // L):
            plist[pl.ds(CH + q * L, L)] = dummyv

        def zero_slice():
            zbase = s * ZSL
            nfull, rem = ZSL // 32, ZSL % 32
            for q in range(nfull):
                pltpu.sync_copy(zbuf, agg.at[pl.ds(zbase + q * 32, 32)])
            if rem:
                pltpu.sync_copy(zbuf.at[pl.ds(0, rem)],
                                agg.at[pl.ds(zbase + nfull * 32, rem)])
        zero_slice()

        def pack_sub(k, _):
            off = base_e + k * SUB
            pltpu.sync_copy(et_h.at[pl.ds(off, SUB)], et_s)
            pltpu.sync_copy(src_h.at[pl.ds(off, SUB)], src_s)
            pltpu.sync_copy(dst_h.at[pl.ds(off, SUB)], dst_s)

            def pack_v(v, _):
                pk = (src_s[pl.ds(v * L, L)]
                      | (dst_s[pl.ds(v * L, L)] << SHIFT_D)
                      | (et_s[pl.ds(v * L, L)] << SHIFT_E))
                plist[pl.ds(k * SUB + v * L, L)] = pk
                return 0
            return lax.fori_loop(0, VPS, pack_v, 0)
        lax.fori_loop(0, NSUB, pack_sub, 0)

        plsc.subcore_barrier()

        for r in range(RPC):
            rg = rel0 + r

            def build(b, gi, di):
                for q in range(BATCH // L):
                    pk = plist[pl.ds(b * BATCH + q * L, L)]
                    m = ((pk >> SHIFT_E) & 15) == rg
                    dump_v = DUMP + ((s * BATCH + q * L
                                      + jnp.arange(L, dtype=jnp.int32))
                                     % (N_SP - N))
                    gi[pl.ds(q * L, L)] = jnp.where(m, pk & MASK14, 0)
                    di[0, pl.ds(q * L, L)] = jnp.where(
                        m, (pk >> SHIFT_D) & MASK14, dump_v)

            def gather(gi, fb, sem):
                return pltpu.async_copy(feat_h.at[gi], fb, sem)

            build(jnp.int32(0), gidx0, didx0)
            gather(gidx0, fbuf0, sem0)

            def pair_body(i, _):
                b0 = i * 2
                build(b0 + 1, gidx1, didx1)
                gather(gidx1, fbuf1, sem1)
                pltpu.make_async_copy(feat_h.at[gidx0], fbuf0, sem0).wait()
                pltpu.sync_copy(fbuf0, agg.at[didx0.at[0]], add=True)
                build(b0 + 2, gidx0, didx0)
                gather(gidx0, fbuf0, sem0)
                pltpu.make_async_copy(feat_h.at[gidx1], fbuf1, sem1).wait()
                pltpu.sync_copy(fbuf1, agg.at[didx1.at[0]], add=True)
                return 0
            lax.fori_loop(0, NPAIR, pair_body, 0)
            pltpu.make_async_copy(feat_h.at[gidx0], fbuf0, sem0).wait()

            plsc.subcore_barrier()

            @pl.when(s < NS - 1)
            def _():
                pltpu.sync_copy(agg.at[pl.ds(s * WB, WB)],
                                out_h.at[pl.ds(rg * N + s * WB, WB)])

            @pl.when(s == NS - 1)
            def _():
                pltpu.sync_copy(agg.at[pl.ds((NS - 1) * WB, WB_LAST)],
                                out_h.at[pl.ds(rg * N + (NS - 1) * WB, WB_LAST)])

            if r < RPC - 1:
                zero_slice()
                plsc.subcore_barrier()

    return sc_kernel(feat, src, dst, et)


def _tc_matmul(agg_flat, weight):
    R, F, O = weight.shape
    N = agg_flat.shape[0] // R
    BN = 1000
    NB = N // BN

    def mm_body(a_ref, w_ref, o_ref):
        r = pl.program_id(1)

        @pl.when(r == 0)
        def _():
            o_ref[...] = jnp.zeros_like(o_ref)
        o_ref[...] += jnp.dot(a_ref[...], w_ref[0],
                              preferred_element_type=jnp.float32)

    return pl.pallas_call(
        mm_body,
        grid=(NB, R),
        in_specs=[
            pl.BlockSpec((BN, F), lambda i, r: (r * NB + i, 0)),
            pl.BlockSpec((1, F, O), lambda i, r: (r, 0, 0)),
        ],
        out_specs=pl.BlockSpec((BN, O), lambda i, r: (i, 0)),
        out_shape=jax.ShapeDtypeStruct((N, O), jnp.float32),
        compiler_params=pltpu.CompilerParams(
            dimension_semantics=("parallel", "arbitrary")),
    )(agg_flat, weight)


def kernel(feat, edge_index, etypes, weight):
    N = feat.shape[0]
    E = etypes.shape[0]
    CH = -(-E // (NS * 1024)) * 1024
    EP = CH * NS
    pad = EP - E
    src = jnp.pad(edge_index[0], (0, pad))
    dst = jnp.pad(edge_index[1], (0, pad), constant_values=N)
    et = jnp.pad(etypes, (0, pad))
    agg = _sc_aggregate(feat, src, dst, et, weight.shape[0])
    return _tc_matmul(agg, weight)


# speedup vs baseline: 73.9365x; 73.9365x over previous
import functools

import jax
import jax.numpy as jnp
from jax import lax
from jax.experimental import pallas as pl
from jax.experimental.pallas import tpu as pltpu
from jax.experimental.pallas import tpu_sc as plsc

NC = 2
NS = 16
L = 16


def _sc_aggregate(feat, src, dst, et, num_rels):
    N, F = feat.shape
    E = et.shape[0]
    RPC = num_rels // NC
    CH = E // NS
    SUB = 1024
    NSUB = CH // SUB
    VPS = SUB // L
    BATCH = 64
    NB = CH // BATCH
    NPAIR = NB // 2
    SHIFT_D = 14
    SHIFT_E = 28
    MASK14 = (1 << 14) - 1
    ZSL = -(-(N + 1) // (NS * 8)) * 8
    N_SP = ZSL * NS
    WB = (N // (NS * 8)) * 8
    WB_LAST = N - (NS - 1) * WB
    DUMP = N

    mesh = plsc.VectorSubcoreMesh(core_axis_name="c", subcore_axis_name="s")

    @functools.partial(
        pl.kernel,
        out_type=jax.ShapeDtypeStruct((num_rels * N, F), jnp.float32),
        mesh=mesh,
        compiler_params=pltpu.CompilerParams(use_tc_tiling_on_sc=False),
        scratch_types=[
            pltpu.VMEM((SUB,), jnp.int32),
            pltpu.VMEM((SUB,), jnp.int32),
            pltpu.VMEM((SUB,), jnp.int32),
            pltpu.VMEM((CH + BATCH,), jnp.int32),
            pltpu.VMEM((BATCH,), jnp.int32),
            pltpu.VMEM((BATCH,), jnp.int32),
            pltpu.VMEM((1, BATCH), jnp.int32),
            pltpu.VMEM((1, BATCH), jnp.int32),
            pltpu.VMEM((BATCH, F), jnp.float32),
            pltpu.VMEM((BATCH, F), jnp.float32),
            pltpu.VMEM((32, F), jnp.float32),
            pltpu.VMEM_SHARED((N_SP, F), jnp.float32),
            pltpu.SemaphoreType.DMA,
            pltpu.SemaphoreType.DMA,
        ],
    )
    def sc_kernel(feat_h, src_h, dst_h, et_h, out_h,
                  et_s, src_s, dst_s, plist, gidx0, gidx1, didx0, didx1,
                  fbuf0, fbuf1, zbuf, agg, sem0, sem1):
        c = lax.axis_index("c")
        s = lax.axis_index("s")
        base_e = s * CH
        rel0 = c * RPC

        zv = jnp.zeros((L,), jnp.float32)
        dummyv = jnp.full((L,), DUMP << SHIFT_D, jnp.int32)
        spl = F // L

        def zb_body(k, _):
            zbuf[k // spl, pl.ds((k % spl) * L, L)] = zv
            return 0
        lax.fori_loop(0, 32 * spl, zb_body, 0)
        for q in range(BATCH // L):
            plist[pl.ds(CH + q * L, L)] = dummyv

        def zero_slice():
            zbase = s * ZSL
            nfull, rem = ZSL // 32, ZSL % 32
            for q in range(nfull):
                pltpu.sync_copy(zbuf, agg.at[pl.ds(zbase + q * 32, 32)])
            if rem:
                pltpu.sync_copy(zbuf.at[pl.ds(0, rem)],
                                agg.at[pl.ds(zbase + nfull * 32, rem)])
        zero_slice()

        def pack_sub(k, _):
            off = base_e + k * SUB
            pltpu.sync_copy(et_h.at[pl.ds(off, SUB)], et_s)
            pltpu.sync_copy(src_h.at[pl.ds(off, SUB)], src_s)
            pltpu.sync_copy(dst_h.at[pl.ds(off, SUB)], dst_s)

            def pack_v(v, _):
                pk = (src_s[pl.ds(v * L, L)]
                      | (dst_s[pl.ds(v * L, L)] << SHIFT_D)
                      | (et_s[pl.ds(v * L, L)] << SHIFT_E))
                plist[pl.ds(k * SUB + v * L, L)] = pk
                return 0
            return lax.fori_loop(0, VPS, pack_v, 0)
        lax.fori_loop(0, NSUB, pack_sub, 0)

        plsc.subcore_barrier()

        for r in range(RPC):
            rg = rel0 + r

            def build(b, gi, di):
                for q in range(BATCH // L):
                    pk = plist[pl.ds(b * BATCH + q * L, L)]
                    m = ((pk >> SHIFT_E) & 15) == rg
                    base_v = (s * BATCH + q * L
                              + jnp.arange(L, dtype=jnp.int32))
                    dump_v = DUMP + base_v % (N_SP - N)
                    gi[pl.ds(q * L, L)] = jnp.where(m, pk & MASK14, base_v)
                    di[0, pl.ds(q * L, L)] = jnp.where(
                        m, (pk >> SHIFT_D) & MASK14, dump_v)

            def gather(gi, fb, sem):
                return pltpu.async_copy(feat_h.at[gi], fb, sem)

            build(jnp.int32(0), gidx0, didx0)
            gather(gidx0, fbuf0, sem0)

            def pair_body(i, _):
                b0 = i * 2
                build(b0 + 1, gidx1, didx1)
                gather(gidx1, fbuf1, sem1)
                pltpu.make_async_copy(feat_h.at[gidx0], fbuf0, sem0).wait()
                pltpu.sync_copy(fbuf0, agg.at[didx0.at[0]], add=True)
                build(b0 + 2, gidx0, didx0)
                gather(gidx0, fbuf0, sem0)
                pltpu.make_async_copy(feat_h.at[gidx1], fbuf1, sem1).wait()
                pltpu.sync_copy(fbuf1, agg.at[didx1.at[0]], add=True)
                return 0
            lax.fori_loop(0, NPAIR, pair_body, 0)
            pltpu.make_async_copy(feat_h.at[gidx0], fbuf0, sem0).wait()

            plsc.subcore_barrier()

            @pl.when(s < NS - 1)
            def _():
                pltpu.sync_copy(agg.at[pl.ds(s * WB, WB)],
                                out_h.at[pl.ds(rg * N + s * WB, WB)])

            @pl.when(s == NS - 1)
            def _():
                pltpu.sync_copy(agg.at[pl.ds((NS - 1) * WB, WB_LAST)],
                                out_h.at[pl.ds(rg * N + (NS - 1) * WB, WB_LAST)])

            if r < RPC - 1:
                zero_slice()
                plsc.subcore_barrier()

    return sc_kernel(feat, src, dst, et)


def _tc_matmul(agg_flat, weight):
    R, F, O = weight.shape
    N = agg_flat.shape[0] // R
    BN = 1000
    NB = N // BN

    def mm_body(a_ref, w_ref, o_ref):
        r = pl.program_id(1)

        @pl.when(r == 0)
        def _():
            o_ref[...] = jnp.zeros_like(o_ref)
        o_ref[...] += jnp.dot(a_ref[...], w_ref[0],
                              preferred_element_type=jnp.float32)

    return pl.pallas_call(
        mm_body,
        grid=(NB, R),
        in_specs=[
            pl.BlockSpec((BN, F), lambda i, r: (r * NB + i, 0)),
            pl.BlockSpec((1, F, O), lambda i, r: (r, 0, 0)),
        ],
        out_specs=pl.BlockSpec((BN, O), lambda i, r: (i, 0)),
        out_shape=jax.ShapeDtypeStruct((N, O), jnp.float32),
        compiler_params=pltpu.CompilerParams(
            dimension_semantics=("parallel", "arbitrary")),
    )(agg_flat, weight)


def kernel(feat, edge_index, etypes, weight):
    N = feat.shape[0]
    E = etypes.shape[0]
    CH = -(-E // (NS * 1024)) * 1024
    EP = CH * NS
    pad = EP - E
    src = jnp.pad(edge_index[0], (0, pad))
    dst = jnp.pad(edge_index[1], (0, pad), constant_values=N)
    et = jnp.pad(etypes, (0, pad))
    agg = _sc_aggregate(feat, src, dst, et, weight.shape[0])
    return _tc_matmul(agg, weight)
